# baseline (speedup 1.0000x reference)
import functools

import jax
import jax.numpy as jnp
from jax import lax
from jax.experimental import pallas as pl
from jax.experimental.pallas import tpu as pltpu
from jax.experimental.pallas import tpu_sc as plsc

N_PTS = 524288
NC, NS, LANES = 2, 16, 16
NW = NC * NS
NPW = N_PTS // NW
C = 512
NV = C // LANES
NCHUNK = NPW // C
GSUB = 128
NSUB = C // GSUB

RES = (128, 256, 512, 1024)
PLANE_PAIRS = ((0, 1), (0, 2), (1, 2))

_OFFS = []
_off = 0
for _pi in range(3):
    for _R in RES:
        _OFFS.append(_off)
        _off += _R * _R
TBL_ROWS = _off

BUILD_NX = {128: 4, 256: 8, 512: 4, 1024: 2}
_OUTB_ROWS = 2048


@functools.partial(
    pl.kernel,
    mesh=plsc.VectorSubcoreMesh(core_axis_name="c", subcore_axis_name="s"),
    out_type=jax.ShapeDtypeStruct((TBL_ROWS, 8), jnp.float32),
    compiler_params=pltpu.CompilerParams(
        needs_layout_passes=False, use_tc_tiling_on_sc=False
    ),
    scratch_types=[
        pltpu.VMEM((BUILD_NX[128] + 2, 128, 2), jnp.float32),
        pltpu.VMEM((BUILD_NX[256] + 2, 256, 2), jnp.float32),
        pltpu.VMEM((BUILD_NX[512] + 2, 512, 2), jnp.float32),
        pltpu.VMEM((BUILD_NX[1024] + 2, 1024, 2), jnp.float32),
        pltpu.VMEM((_OUTB_ROWS, 8), jnp.float32),
    ],
)
def _sc_table_builder(*refs):
    grids = refs[:12]
    tbl = refs[12]
    strips = refs[13:17]
    outb = refs[17]
    wid = lax.axis_index("s") * NC + lax.axis_index("c")
    iota = lax.iota(jnp.int32, LANES)
    j = iota & 7
    rip = iota >> 3
    hi = (iota >> 2) & 1
    yoff = (iota >> 1) & 1
    fidx = iota & 1

    for combo in range(12):
        level = combo % 4
        R = RES[level]
        NX = BUILD_NX[R]
        XPW = R // NW
        NB = XPW // NX
        g = grids[combo]
        strip = strips[level]
        x0w = wid * XPW

        def block_body(blk, carry, g=g, strip=strip, R=R, NX=NX,
                       x0w=x0w, combo=combo):
            xb = x0w + blk * NX
            xs = jnp.minimum(xb, R - NX - 1)
            pltpu.sync_copy(g.at[pl.ds(xs, NX + 1), :, :],
                            strip.at[pl.ds(0, NX + 1), :, :])
            xrel0 = xb - xs

            def xrow_body(xr, carry2, strip=strip, R=R, xrel0=xrel0):
                xi = xrel0 + xr + hi

                def yv_body(vy, carry3, strip=strip, R=R, xr=xr, xi=xi):
                    yi = 2 * vy + (rip + yoff)
                    val = plsc.load_gather(strip, [xi, yi, fidx])
                    ri = (xr * R // 2 + vy) * 2 + rip
                    plsc.store_scatter(outb, [ri, j], val)
                    return carry3

                lax.fori_loop(0, R // 2, yv_body, 0)
                return carry2

            lax.fori_loop(0, NX, xrow_body, 0)
            pltpu.sync_copy(
                outb.at[pl.ds(0, NX * R), :],
                tbl.at[pl.ds(_OFFS[combo] + xb * R, NX * R), :],
            )
            return carry

        lax.fori_loop(0, NB, block_body, 0)


@functools.partial(
    pl.kernel,
    mesh=plsc.VectorSubcoreMesh(core_axis_name="c", subcore_axis_name="s"),
    out_type=jax.ShapeDtypeStruct((N_PTS, 24), jnp.float32),
    compiler_params=pltpu.CompilerParams(
        needs_layout_passes=False, use_tc_tiling_on_sc=False
    ),
    scratch_types=[
        pltpu.VMEM((C, 3), jnp.float32),
        pltpu.VMEM((12 * C,), jnp.float32),
        pltpu.VMEM((12 * C,), jnp.int32),
        pltpu.VMEM((12 * C,), jnp.int32),
        pltpu.VMEM((C, 8), jnp.float32),
        pltpu.VMEM((C, 24), jnp.float32),
        pltpu.VMEM((2 * LANES,), jnp.float32),
        pltpu.SemaphoreType.DMA,
    ],
)
def _sc_plane_kernel(x2d, tbl, par, out_hbm, xq, fr, i0r, idxr, rows, outb, parv, sem):
    wid = lax.axis_index("s") * NC + lax.axis_index("c")
    pltpu.sync_copy(par, parv)
    bv = parv[pl.ds(0, LANES)]
    inv = parv[pl.ds(LANES, LANES)]
    iota = lax.iota(jnp.int32, LANES)

    def chunk_body(ch, carry):
        base = wid * NPW + ch * C
        pltpu.sync_copy(x2d.at[pl.ds(base, C), :], xq)

        def coord_body(v, carry2):
            off16 = v * LANES
            pt = iota + off16
            for a in range(3):
                xv = plsc.load_gather(xq, [pt, jnp.full((LANES,), a, jnp.int32)])
                xn = jnp.clip((xv + bv) * inv, 0.0, 1.0)
                for l, R in enumerate(RES):
                    p = xn * (R - 1)
                    i0 = jnp.minimum(p.astype(jnp.int32), R - 2)
                    i0r[pl.ds((a * 4 + l) * C + off16, LANES)] = i0
                    fr[pl.ds((a * 4 + l) * C + off16, LANES)] = p - i0.astype(jnp.float32)
            return carry2

        lax.fori_loop(0, NV, coord_body, 0)

        def idx_body(v, carry2):
            off16 = v * LANES
            for pi, (a, b) in enumerate(PLANE_PAIRS):
                for l, R in enumerate(RES):
                    combo = pi * 4 + l
                    xi = i0r[pl.ds((a * 4 + l) * C + off16, LANES)]
                    yi = i0r[pl.ds((b * 4 + l) * C + off16, LANES)]
                    idxr[pl.ds(combo * C + off16, LANES)] = xi * R + yi + _OFFS[combo]
            return carry2

        lax.fori_loop(0, NV, idx_body, 0)

        for pi, (a, b) in enumerate(PLANE_PAIRS):
            for l in range(4):
                combo = pi * 4 + l
                copies = [
                    pltpu.async_copy(
                        tbl.at[idxr.at[pl.ds(combo * C + jj * GSUB, GSUB)]],
                        rows.at[pl.ds(jj * GSUB, GSUB), :],
                        sem,
                    )
                    for jj in range(NSUB)
                ]
                for cp in copies:
                    cp.wait()

                fxoff = (a * 4 + l) * C
                fyoff = (b * 4 + l) * C

                def interp_body(v, carry2, fxoff=fxoff, fyoff=fyoff, combo=combo):
                    off16 = v * LANES
                    pt = iota + off16
                    fx = fr[pl.ds(fxoff + off16, LANES)]
                    fy = fr[pl.ds(fyoff + off16, LANES)]
                    g = [
                        plsc.load_gather(rows, [pt, jnp.full((LANES,), col, jnp.int32)])
                        for col in range(8)
                    ]
                    for ff in range(2):
                        a0 = g[ff] + fy * (g[2 + ff] - g[ff])
                        a1 = g[4 + ff] + fy * (g[6 + ff] - g[4 + ff])
                        o = a0 + fx * (a1 - a0)
                        plsc.store_scatter(
                            outb, [pt, jnp.full((LANES,), 2 * combo + ff, jnp.int32)], o
                        )
                    return carry2

                lax.fori_loop(0, NV, interp_body, 0)

        pltpu.sync_copy(outb, out_hbm.at[pl.ds(base, C), :])
        return carry

    lax.fori_loop(0, NCHUNK, chunk_body, 0)


def kernel(x, bound,
           xy_g0, xy_g1, xy_g2, xy_g3,
           yz_g0, yz_g1, yz_g2, yz_g3,
           xz_g0, xz_g1, xz_g2, xz_g3):
    grids = [xy_g0, xy_g1, xy_g2, xy_g3,
             yz_g0, yz_g1, yz_g2, yz_g3,
             xz_g0, xz_g1, xz_g2, xz_g3]
    tbl = _sc_table_builder(*grids)
    b = jnp.asarray(bound, jnp.float32)
    par = jnp.concatenate([jnp.full((LANES,), b, jnp.float32),
                           jnp.full((LANES,), 0.5 / b, jnp.float32)])
    return _sc_plane_kernel(x, tbl, par)


# speedup vs baseline: 3.2217x; 3.2217x over previous
import functools

import jax
import jax.numpy as jnp
from jax import lax
from jax.experimental import pallas as pl
from jax.experimental.pallas import tpu as pltpu
from jax.experimental.pallas import tpu_sc as plsc

N_PTS = 524288
NC, NS, LANES = 2, 16, 16
NW = NC * NS
NPW = N_PTS // NW
C = 1024
NV = C // LANES
NCHUNK = NPW // C
GSUB = 128
NSUB = C // GSUB

RES = (128, 256, 512, 1024)
PLANE_PAIRS = ((0, 1), (0, 2), (1, 2))

_OFFS = []
_off = 0
for _pi in range(3):
    for _R in RES:
        _OFFS.append(_off)
        _off += _R * _R
TBL_ROWS = _off

BUILD_NX = {128: 4, 256: 8, 512: 4, 1024: 2}
_OUTB_ROWS = 2048


@functools.partial(
    pl.kernel,
    mesh=plsc.VectorSubcoreMesh(core_axis_name="c", subcore_axis_name="s"),
    out_type=jax.ShapeDtypeStruct((TBL_ROWS, 8), jnp.float32),
    compiler_params=pltpu.CompilerParams(
        needs_layout_passes=False, use_tc_tiling_on_sc=False
    ),
    scratch_types=[
        pltpu.VMEM((BUILD_NX[128] + 2, 2 * 128), jnp.float32),
        pltpu.VMEM((BUILD_NX[256] + 2, 2 * 256), jnp.float32),
        pltpu.VMEM((BUILD_NX[512] + 2, 2 * 512), jnp.float32),
        pltpu.VMEM((BUILD_NX[1024] + 2, 2 * 1024), jnp.float32),
        pltpu.VMEM((_OUTB_ROWS, 8), jnp.float32),
    ],
)
def _sc_table_builder(*refs):
    grids = refs[:12]
    tbl = refs[12]
    strips = refs[13:17]
    outb = refs[17]
    wid = lax.axis_index("s") * NC + lax.axis_index("c")
    iota = lax.iota(jnp.int32, LANES)
    j = iota & 7
    rip = iota >> 3
    hi = (iota >> 2) & 1
    yoff = (iota >> 1) & 1
    fidx = iota & 1

    for combo in range(12):
        level = combo % 4
        R = RES[level]
        NX = BUILD_NX[R]
        XPW = R // NW
        NB = XPW // NX
        g = grids[combo]
        strip = strips[level]
        x0w = wid * XPW

        def block_body(blk, carry, g=g, strip=strip, R=R, NX=NX,
                       x0w=x0w, combo=combo):
            xb = x0w + blk * NX
            xs = jnp.minimum(xb, R - NX - 1)
            pltpu.sync_copy(g.at[pl.ds(xs, NX + 1), :],
                            strip.at[pl.ds(0, NX + 1), :])
            xrel0 = xb - xs

            def xrow_body(xr, carry2, strip=strip, R=R, xrel0=xrel0):
                xi = xrel0 + xr + hi

                def yv_body(vy, carry3, strip=strip, R=R, xr=xr, xi=xi):
                    ci = 4 * vy + (2 * rip + 2 * yoff + fidx)
                    val = plsc.load_gather(strip, [xi, ci])
                    ri = (xr * R // 2 + vy) * 2 + rip
                    plsc.store_scatter(outb, [ri, j], val)
                    return carry3

                lax.fori_loop(0, R // 2, yv_body, 0)
                return carry2

            lax.fori_loop(0, NX, xrow_body, 0)
            pltpu.sync_copy(
                outb.at[pl.ds(0, NX * R), :],
                tbl.at[pl.ds(_OFFS[combo] + xb * R, NX * R), :],
            )
            return carry

        lax.fori_loop(0, NB, block_body, 0)


@functools.partial(
    pl.kernel,
    mesh=plsc.VectorSubcoreMesh(core_axis_name="c", subcore_axis_name="s"),
    out_type=jax.ShapeDtypeStruct((N_PTS, 24), jnp.float32),
    compiler_params=pltpu.CompilerParams(
        needs_layout_passes=False, use_tc_tiling_on_sc=False
    ),
    scratch_types=[
        pltpu.VMEM((3 * C,), jnp.float32),
        pltpu.VMEM((12 * C,), jnp.float32),
        pltpu.VMEM((12 * C,), jnp.int32),
        pltpu.VMEM((12 * C,), jnp.int32),
        pltpu.VMEM((C, 8), jnp.float32),
        pltpu.VMEM((C, 24), jnp.float32),
        pltpu.VMEM((2 * LANES,), jnp.float32),
        pltpu.SemaphoreType.DMA,
    ],
)
def _sc_plane_kernel(x1d, tbl, par, out_hbm, xq, fr, i0r, idxr, rows, outb, parv, sem):
    wid = lax.axis_index("s") * NC + lax.axis_index("c")
    pltpu.sync_copy(par, parv)
    bv = parv[pl.ds(0, LANES)]
    inv = parv[pl.ds(LANES, LANES)]
    iota = lax.iota(jnp.int32, LANES)
    iota3 = iota * 3

    def chunk_body(ch, carry):
        base = wid * NPW + ch * C
        pltpu.sync_copy(x1d.at[pl.ds(3 * base, 3 * C)], xq)

        def coord_body(v, carry2):
            off16 = v * LANES
            for a in range(3):
                xv = plsc.load_gather(xq, [iota3 + (3 * off16 + a)])
                xn = jnp.clip((xv + bv) * inv, 0.0, 1.0)
                for l, R in enumerate(RES):
                    p = xn * (R - 1)
                    i0 = jnp.minimum(p.astype(jnp.int32), R - 2)
                    i0r[pl.ds((a * 4 + l) * C + off16, LANES)] = i0
                    fr[pl.ds((a * 4 + l) * C + off16, LANES)] = p - i0.astype(jnp.float32)
            return carry2

        lax.fori_loop(0, NV, coord_body, 0)

        def idx_body(v, carry2):
            off16 = v * LANES
            for pi, (a, b) in enumerate(PLANE_PAIRS):
                for l, R in enumerate(RES):
                    combo = pi * 4 + l
                    xi = i0r[pl.ds((a * 4 + l) * C + off16, LANES)]
                    yi = i0r[pl.ds((b * 4 + l) * C + off16, LANES)]
                    idxr[pl.ds(combo * C + off16, LANES)] = xi * R + yi + _OFFS[combo]
            return carry2

        lax.fori_loop(0, NV, idx_body, 0)

        for pi, (a, b) in enumerate(PLANE_PAIRS):
            for l in range(4):
                combo = pi * 4 + l
                copies = [
                    pltpu.async_copy(
                        tbl.at[idxr.at[pl.ds(combo * C + jj * GSUB, GSUB)]],
                        rows.at[pl.ds(jj * GSUB, GSUB), :],
                        sem,
                    )
                    for jj in range(NSUB)
                ]
                for cp in copies:
                    cp.wait()

                fxoff = (a * 4 + l) * C
                fyoff = (b * 4 + l) * C

                def interp_body(v, carry2, fxoff=fxoff, fyoff=fyoff, combo=combo):
                    off16 = v * LANES
                    pt = iota + off16
                    fx = fr[pl.ds(fxoff + off16, LANES)]
                    fy = fr[pl.ds(fyoff + off16, LANES)]
                    g = [
                        plsc.load_gather(rows, [pt, jnp.full((LANES,), col, jnp.int32)])
                        for col in range(8)
                    ]
                    for ff in range(2):
                        a0 = g[ff] + fy * (g[2 + ff] - g[ff])
                        a1 = g[4 + ff] + fy * (g[6 + ff] - g[4 + ff])
                        o = a0 + fx * (a1 - a0)
                        plsc.store_scatter(
                            outb, [pt, jnp.full((LANES,), 2 * combo + ff, jnp.int32)], o
                        )
                    return carry2

                lax.fori_loop(0, NV, interp_body, 0)

        pltpu.sync_copy(outb, out_hbm.at[pl.ds(base, C), :])
        return carry

    lax.fori_loop(0, NCHUNK, chunk_body, 0)


def kernel(x, bound,
           xy_g0, xy_g1, xy_g2, xy_g3,
           yz_g0, yz_g1, yz_g2, yz_g3,
           xz_g0, xz_g1, xz_g2, xz_g3):
    grids = [xy_g0, xy_g1, xy_g2, xy_g3,
             yz_g0, yz_g1, yz_g2, yz_g3,
             xz_g0, xz_g1, xz_g2, xz_g3]
    R2V = [g.reshape(g.shape[0], -1) for g in grids]
    tbl = _sc_table_builder(*R2V)
    b = jnp.asarray(bound, jnp.float32)
    par = jnp.concatenate([jnp.full((LANES,), b, jnp.float32),
                           jnp.full((LANES,), 0.5 / b, jnp.float32)])
    return _sc_plane_kernel(x.reshape(-1), tbl, par)


# speedup vs baseline: 4.1196x; 1.2787x over previous
import functools

import jax
import jax.numpy as jnp
from jax import lax
from jax.experimental import pallas as pl
from jax.experimental.pallas import tpu as pltpu
from jax.experimental.pallas import tpu_sc as plsc

N_PTS = 524288
NC, NS, LANES = 2, 16, 16
NW = NC * NS
NPW = N_PTS // NW
C = 1024
NV = C // LANES
NCHUNK = NPW // C
GSUB = 128
NSUB = C // GSUB

RES = (128, 256, 512, 1024)
PLANE_PAIRS = ((0, 1), (0, 2), (1, 2))

_OFFS = []
_off = 0
for _pi in range(3):
    for _R in RES:
        _OFFS.append(_off)
        _off += _R * _R
TBL_ROWS = _off

BUILD_NX = {128: 4, 256: 8, 512: 4, 1024: 2}
_OUTB_ROWS = 2048


@functools.partial(
    pl.kernel,
    mesh=plsc.VectorSubcoreMesh(core_axis_name="c", subcore_axis_name="s"),
    out_type=jax.ShapeDtypeStruct((TBL_ROWS, 8), jnp.float32),
    compiler_params=pltpu.CompilerParams(
        needs_layout_passes=False, use_tc_tiling_on_sc=False
    ),
    scratch_types=[
        pltpu.VMEM((BUILD_NX[128] + 2, 2 * 128), jnp.float32),
        pltpu.VMEM((BUILD_NX[256] + 2, 2 * 256), jnp.float32),
        pltpu.VMEM((BUILD_NX[512] + 2, 2 * 512), jnp.float32),
        pltpu.VMEM((BUILD_NX[1024] + 2, 2 * 1024), jnp.float32),
        pltpu.VMEM((_OUTB_ROWS, 8), jnp.float32),
    ],
)
def _sc_table_builder(*refs):
    grids = refs[:12]
    tbl = refs[12]
    strips = refs[13:17]
    outb = refs[17]
    wid = lax.axis_index("s") * NC + lax.axis_index("c")
    iota = lax.iota(jnp.int32, LANES)
    j = iota & 7
    rip = iota >> 3
    hi = (iota >> 2) & 1
    yoff = (iota >> 1) & 1
    fidx = iota & 1

    for combo in range(12):
        level = combo % 4
        R = RES[level]
        NX = BUILD_NX[R]
        XPW = R // NW
        NB = XPW // NX
        g = grids[combo]
        strip = strips[level]
        x0w = wid * XPW

        def block_body(blk, carry, g=g, strip=strip, R=R, NX=NX,
                       x0w=x0w, combo=combo):
            xb = x0w + blk * NX
            xs = jnp.minimum(xb, R - NX - 1)
            pltpu.sync_copy(g.at[pl.ds(xs, NX + 1), :],
                            strip.at[pl.ds(0, NX + 1), :])
            xrel0 = xb - xs

            def xrow_body(xr, carry2, strip=strip, R=R, xrel0=xrel0):
                xi = xrel0 + xr + hi

                def yv_body(vy, carry3, strip=strip, R=R, xr=xr, xi=xi):
                    ci = 2 * vy + (rip + yoff + fidx * R)
                    val = plsc.load_gather(strip, [xi, ci])
                    ri = (xr * R // 2 + vy) * 2 + rip
                    plsc.store_scatter(outb, [ri, j], val)
                    return carry3

                lax.fori_loop(0, R // 2, yv_body, 0)
                return carry2

            lax.fori_loop(0, NX, xrow_body, 0)
            pltpu.sync_copy(
                outb.at[pl.ds(0, NX * R), :],
                tbl.at[pl.ds(_OFFS[combo] + xb * R, NX * R), :],
            )
            return carry

        lax.fori_loop(0, NB, block_body, 0)


@functools.partial(
    pl.kernel,
    mesh=plsc.VectorSubcoreMesh(core_axis_name="c", subcore_axis_name="s"),
    out_type=jax.ShapeDtypeStruct((N_PTS, 24), jnp.float32),
    compiler_params=pltpu.CompilerParams(
        needs_layout_passes=False, use_tc_tiling_on_sc=False
    ),
    scratch_types=[
        pltpu.VMEM((3 * C,), jnp.float32),
        pltpu.VMEM((12 * C,), jnp.float32),
        pltpu.VMEM((12 * C,), jnp.int32),
        pltpu.VMEM((12 * C,), jnp.int32),
        pltpu.VMEM((C, 8), jnp.float32),
        pltpu.VMEM((C, 24), jnp.float32),
        pltpu.VMEM((2 * LANES,), jnp.float32),
        pltpu.SemaphoreType.DMA,
    ],
)
def _sc_plane_kernel(x1d, tbl, par, out_hbm, xq, fr, i0r, idxr, rows, outb, parv, sem):
    wid = lax.axis_index("s") * NC + lax.axis_index("c")
    pltpu.sync_copy(par, parv)
    bv = parv[pl.ds(0, LANES)]
    inv = parv[pl.ds(LANES, LANES)]
    iota = lax.iota(jnp.int32, LANES)

    def chunk_body(ch, carry):
        base = wid * NPW + ch * C
        for a in range(3):
            pltpu.sync_copy(x1d.at[pl.ds(a * N_PTS + base, C)],
                            xq.at[pl.ds(a * C, C)])

        def coord_body(v, carry2):
            off16 = v * LANES
            for a in range(3):
                xv = xq[pl.ds(a * C + off16, LANES)]
                xn = jnp.clip((xv + bv) * inv, 0.0, 1.0)
                for l, R in enumerate(RES):
                    p = xn * (R - 1)
                    i0 = jnp.minimum(p.astype(jnp.int32), R - 2)
                    i0r[pl.ds((a * 4 + l) * C + off16, LANES)] = i0
                    fr[pl.ds((a * 4 + l) * C + off16, LANES)] = p - i0.astype(jnp.float32)
            return carry2

        lax.fori_loop(0, NV, coord_body, 0)

        def idx_body(v, carry2):
            off16 = v * LANES
            for pi, (a, b) in enumerate(PLANE_PAIRS):
                for l, R in enumerate(RES):
                    combo = pi * 4 + l
                    xi = i0r[pl.ds((a * 4 + l) * C + off16, LANES)]
                    yi = i0r[pl.ds((b * 4 + l) * C + off16, LANES)]
                    idxr[pl.ds(combo * C + off16, LANES)] = xi * R + yi + _OFFS[combo]
            return carry2

        lax.fori_loop(0, NV, idx_body, 0)

        for pi, (a, b) in enumerate(PLANE_PAIRS):
            for l in range(4):
                combo = pi * 4 + l
                copies = [
                    pltpu.async_copy(
                        tbl.at[idxr.at[pl.ds(combo * C + jj * GSUB, GSUB)]],
                        rows.at[pl.ds(jj * GSUB, GSUB), :],
                        sem,
                    )
                    for jj in range(NSUB)
                ]
                for cp in copies:
                    cp.wait()

                fxoff = (a * 4 + l) * C
                fyoff = (b * 4 + l) * C

                def interp_body(v, carry2, fxoff=fxoff, fyoff=fyoff, combo=combo):
                    off16 = v * LANES
                    pt = iota + off16
                    fx = fr[pl.ds(fxoff + off16, LANES)]
                    fy = fr[pl.ds(fyoff + off16, LANES)]
                    g = [
                        plsc.load_gather(rows, [pt, jnp.full((LANES,), col, jnp.int32)])
                        for col in range(8)
                    ]
                    for ff in range(2):
                        a0 = g[ff] + fy * (g[2 + ff] - g[ff])
                        a1 = g[4 + ff] + fy * (g[6 + ff] - g[4 + ff])
                        o = a0 + fx * (a1 - a0)
                        plsc.store_scatter(
                            outb, [pt, jnp.full((LANES,), 2 * combo + ff, jnp.int32)], o
                        )
                    return carry2

                lax.fori_loop(0, NV, interp_body, 0)

        pltpu.sync_copy(outb, out_hbm.at[pl.ds(base, C), :])
        return carry

    lax.fori_loop(0, NCHUNK, chunk_body, 0)


def kernel(x, bound,
           xy_g0, xy_g1, xy_g2, xy_g3,
           yz_g0, yz_g1, yz_g2, yz_g3,
           xz_g0, xz_g1, xz_g2, xz_g3):
    grids = [xy_g0, xy_g1, xy_g2, xy_g3,
             yz_g0, yz_g1, yz_g2, yz_g3,
             xz_g0, xz_g1, xz_g2, xz_g3]
    R2V = [g.transpose(0, 2, 1).reshape(g.shape[0], -1) for g in grids]
    tbl = _sc_table_builder(*R2V)
    b = jnp.asarray(bound, jnp.float32)
    par = jnp.concatenate([jnp.full((LANES,), b, jnp.float32),
                           jnp.full((LANES,), 0.5 / b, jnp.float32)])
    return _sc_plane_kernel(x.T.reshape(-1), tbl, par)


# speedup vs baseline: 5.9189x; 1.4367x over previous
import functools

import jax
import jax.numpy as jnp
from jax import lax
from jax.experimental import pallas as pl
from jax.experimental.pallas import tpu as pltpu
from jax.experimental.pallas import tpu_sc as plsc

N_PTS = 524288
NC, NS, LANES = 2, 16, 16
NW = NC * NS
NPW = N_PTS // NW
C = 1024
NV = C // LANES
NCHUNK = NPW // C
GSUB = 128
NSUB = C // GSUB

RES = (128, 256, 512, 1024)
PLANE_PAIRS = ((0, 1), (0, 2), (1, 2))

_OFFS = []
_off = 0
for _pi in range(3):
    for _R in RES:
        _OFFS.append(_off)
        _off += _R * _R
TBL_ROWS = _off

BUILD_NX = {128: 4, 256: 8, 512: 4, 1024: 2}
_OUTB_ROWS = 2048


@functools.partial(
    pl.kernel,
    mesh=plsc.VectorSubcoreMesh(core_axis_name="c", subcore_axis_name="s"),
    out_type=jax.ShapeDtypeStruct((TBL_ROWS, 8), jnp.float32),
    compiler_params=pltpu.CompilerParams(
        needs_layout_passes=False, use_tc_tiling_on_sc=False
    ),
    scratch_types=[
        pltpu.VMEM((BUILD_NX[128] + 2, 2 * 128), jnp.float32),
        pltpu.VMEM((BUILD_NX[256] + 2, 2 * 256), jnp.float32),
        pltpu.VMEM((BUILD_NX[512] + 2, 2 * 512), jnp.float32),
        pltpu.VMEM((BUILD_NX[1024] + 2, 2 * 1024), jnp.float32),
        pltpu.VMEM((_OUTB_ROWS, 8), jnp.float32),
    ],
)
def _sc_table_builder(*refs):
    grids = refs[:12]
    tbl = refs[12]
    strips = refs[13:17]
    outb = refs[17]
    wid = lax.axis_index("s") * NC + lax.axis_index("c")
    iota = lax.iota(jnp.int32, LANES)
    j = iota & 7
    rip = iota >> 3
    hi = (iota >> 2) & 1
    yoff = (iota >> 1) & 1
    fidx = iota & 1

    for combo in range(12):
        level = combo % 4
        R = RES[level]
        NX = BUILD_NX[R]
        XPW = R // NW
        NB = XPW // NX
        g = grids[combo]
        strip = strips[level]
        x0w = wid * XPW

        def block_body(blk, carry, g=g, strip=strip, R=R, NX=NX,
                       x0w=x0w, combo=combo):
            xb = x0w + blk * NX
            xs = jnp.minimum(xb, R - NX - 1)
            pltpu.sync_copy(g.at[pl.ds(xs, NX + 1), :],
                            strip.at[pl.ds(0, NX + 1), :])
            xrel0 = xb - xs

            def xrow_body(xr, carry2, strip=strip, R=R, xrel0=xrel0):
                xi = xrel0 + xr + hi

                def yv_body(vy, carry3, strip=strip, R=R, xr=xr, xi=xi):
                    ci = 4 * vy + (2 * rip + 2 * yoff + fidx)
                    val = plsc.load_gather(strip, [xi, ci])
                    ri = (xr * R // 2 + vy) * 2 + rip
                    plsc.store_scatter(outb, [ri, j], val)
                    return carry3

                lax.fori_loop(0, R // 2, yv_body, 0)
                return carry2

            lax.fori_loop(0, NX, xrow_body, 0)
            pltpu.sync_copy(
                outb.at[pl.ds(0, NX * R), :],
                tbl.at[pl.ds(_OFFS[combo] + xb * R, NX * R), :],
            )
            return carry

        lax.fori_loop(0, NB, block_body, 0)


@functools.partial(
    pl.kernel,
    mesh=plsc.VectorSubcoreMesh(core_axis_name="c", subcore_axis_name="s"),
    out_type=jax.ShapeDtypeStruct((24, N_PTS), jnp.float32),
    compiler_params=pltpu.CompilerParams(
        needs_layout_passes=False, use_tc_tiling_on_sc=False
    ),
    scratch_types=[
        pltpu.VMEM((3 * C,), jnp.float32),
        pltpu.VMEM((12 * C,), jnp.float32),
        pltpu.VMEM((12 * C,), jnp.int32),
        pltpu.VMEM((12 * C,), jnp.int32),
        pltpu.VMEM((C, 8), jnp.float32),
        pltpu.VMEM((C, 8), jnp.float32),
        pltpu.VMEM((24, C), jnp.float32),
        pltpu.VMEM((2 * LANES,), jnp.float32),
        pltpu.SemaphoreType.DMA,
        pltpu.SemaphoreType.DMA,
    ],
)
def _sc_plane_kernel(x1d, tbl, par, out_hbm, xq, fr, i0r, idxr,
                     rows_a, rows_b, outb, parv, sem_a, sem_b):
    wid = lax.axis_index("s") * NC + lax.axis_index("c")
    pltpu.sync_copy(par, parv)
    bv = parv[pl.ds(0, LANES)]
    inv = parv[pl.ds(LANES, LANES)]
    iota = lax.iota(jnp.int32, LANES)

    def chunk_body(ch, carry):
        base = wid * NPW + ch * C
        for a in range(3):
            pltpu.sync_copy(x1d.at[pl.ds(a * N_PTS + base, C)],
                            xq.at[pl.ds(a * C, C)])

        def coord_body(v, carry2):
            off16 = v * LANES
            for a in range(3):
                xv = xq[pl.ds(a * C + off16, LANES)]
                xn = jnp.clip((xv + bv) * inv, 0.0, 1.0)
                for l, R in enumerate(RES):
                    p = xn * (R - 1)
                    i0 = jnp.minimum(p.astype(jnp.int32), R - 2)
                    i0r[pl.ds((a * 4 + l) * C + off16, LANES)] = i0
                    fr[pl.ds((a * 4 + l) * C + off16, LANES)] = p - i0.astype(jnp.float32)
            return carry2

        lax.fori_loop(0, NV, coord_body, 0)

        def idx_body(v, carry2):
            off16 = v * LANES
            for pi, (a, b) in enumerate(PLANE_PAIRS):
                for l, R in enumerate(RES):
                    combo = pi * 4 + l
                    xi = i0r[pl.ds((a * 4 + l) * C + off16, LANES)]
                    yi = i0r[pl.ds((b * 4 + l) * C + off16, LANES)]
                    idxr[pl.ds(combo * C + off16, LANES)] = xi * R + yi + _OFFS[combo]
            return carry2

        lax.fori_loop(0, NV, idx_body, 0)

        bufs = (rows_a, rows_b)
        sems = (sem_a, sem_b)

        def fire(combo):
            rows, sem = bufs[combo % 2], sems[combo % 2]
            return [
                pltpu.async_copy(
                    tbl.at[idxr.at[pl.ds(combo * C + jj * GSUB, GSUB)]],
                    rows.at[pl.ds(jj * GSUB, GSUB), :],
                    sem,
                )
                for jj in range(NSUB)
            ]

        inflight = fire(0)
        for pi, (a, b) in enumerate(PLANE_PAIRS):
            for l in range(4):
                combo = pi * 4 + l
                rows = bufs[combo % 2]
                for cp in inflight:
                    cp.wait()
                if combo + 1 < 12:
                    inflight = fire(combo + 1)

                fxoff = (a * 4 + l) * C
                fyoff = (b * 4 + l) * C

                def interp_body(v, carry2, rows=rows, fxoff=fxoff,
                                fyoff=fyoff, combo=combo):
                    off16 = v * LANES
                    pt = iota + off16
                    fx = fr[pl.ds(fxoff + off16, LANES)]
                    fy = fr[pl.ds(fyoff + off16, LANES)]
                    g = [
                        plsc.load_gather(rows, [pt, jnp.full((LANES,), col, jnp.int32)])
                        for col in range(8)
                    ]
                    for ff in range(2):
                        a0 = g[ff] + fy * (g[2 + ff] - g[ff])
                        a1 = g[4 + ff] + fy * (g[6 + ff] - g[4 + ff])
                        o = a0 + fx * (a1 - a0)
                        outb[2 * combo + ff, pl.ds(off16, LANES)] = o
                    return carry2

                lax.fori_loop(0, NV, interp_body, 0)

        pltpu.sync_copy(outb, out_hbm.at[:, pl.ds(base, C)])
        return carry

    lax.fori_loop(0, NCHUNK, chunk_body, 0)


def kernel(x, bound,
           xy_g0, xy_g1, xy_g2, xy_g3,
           yz_g0, yz_g1, yz_g2, yz_g3,
           xz_g0, xz_g1, xz_g2, xz_g3):
    grids = [xy_g0, xy_g1, xy_g2, xy_g3,
             yz_g0, yz_g1, yz_g2, yz_g3,
             xz_g0, xz_g1, xz_g2, xz_g3]
    R2V = [g.reshape(g.shape[0], -1) for g in grids]
    tbl = _sc_table_builder(*R2V)
    b = jnp.asarray(bound, jnp.float32)
    par = jnp.concatenate([jnp.full((LANES,), b, jnp.float32),
                           jnp.full((LANES,), 0.5 / b, jnp.float32)])
    return _sc_plane_kernel(x.T.reshape(-1), tbl, par).T


# speedup vs baseline: 5.9328x; 1.0023x over previous
import functools

import jax
import jax.numpy as jnp
from jax import lax
from jax.experimental import pallas as pl
from jax.experimental.pallas import tpu as pltpu
from jax.experimental.pallas import tpu_sc as plsc

N_PTS = 524288
NC, NS, LANES = 2, 16, 16
NW = NC * NS
NPW = N_PTS // NW
C = 1024
NV = C // LANES
NCHUNK = NPW // C
GSUB = 128
NSUB = C // GSUB

RES = (128, 256, 512, 1024)
PLANE_PAIRS = ((0, 1), (0, 2), (1, 2))

_OFFS = []
_off = 0
for _pi in range(3):
    for _R in RES:
        _OFFS.append(_off)
        _off += _R * _R
TBL_ROWS = _off

BUILD_NX = {128: 4, 256: 8, 512: 4, 1024: 2}
_OUTB_ROWS = 2048


@functools.partial(
    pl.kernel,
    mesh=plsc.VectorSubcoreMesh(core_axis_name="c", subcore_axis_name="s"),
    out_type=jax.ShapeDtypeStruct((TBL_ROWS, 8), jnp.float32),
    compiler_params=pltpu.CompilerParams(
        needs_layout_passes=False, use_tc_tiling_on_sc=False
    ),
    scratch_types=[
        pltpu.VMEM((BUILD_NX[128] + 2, 2 * 128), jnp.float32),
        pltpu.VMEM((BUILD_NX[256] + 2, 2 * 256), jnp.float32),
        pltpu.VMEM((BUILD_NX[512] + 2, 2 * 512), jnp.float32),
        pltpu.VMEM((BUILD_NX[1024] + 2, 2 * 1024), jnp.float32),
        pltpu.VMEM((_OUTB_ROWS, 8), jnp.float32),
    ],
)
def _sc_table_builder(*refs):
    grids = refs[:12]
    tbl = refs[12]
    strips = refs[13:17]
    outb = refs[17]
    wid = lax.axis_index("s") * NC + lax.axis_index("c")
    iota = lax.iota(jnp.int32, LANES)
    j = iota & 7
    rip = iota >> 3
    hi = (iota >> 2) & 1
    yoff = (iota >> 1) & 1
    fidx = iota & 1

    for combo in range(12):
        level = combo % 4
        R = RES[level]
        NX = BUILD_NX[R]
        XPW = R // NW
        NB = XPW // NX
        g = grids[combo]
        strip = strips[level]
        x0w = wid * XPW

        def block_body(blk, carry, g=g, strip=strip, R=R, NX=NX,
                       x0w=x0w, combo=combo):
            xb = x0w + blk * NX
            xs = jnp.minimum(xb, R - NX - 1)
            pltpu.sync_copy(g.at[pl.ds(xs, NX + 1), :],
                            strip.at[pl.ds(0, NX + 1), :])
            xrel0 = xb - xs

            def xrow_body(xr, carry2, strip=strip, R=R, xrel0=xrel0):
                xi = xrel0 + xr + hi

                def yv_body(vy, carry3, strip=strip, R=R, xr=xr, xi=xi):
                    ci = 4 * vy + (2 * rip + 2 * yoff + fidx)
                    val = plsc.load_gather(strip, [xi, ci])
                    ri = (xr * R // 2 + vy) * 2 + rip
                    plsc.store_scatter(outb, [ri, j], val)
                    return carry3

                lax.fori_loop(0, R // 2, yv_body, 0)
                return carry2

            lax.fori_loop(0, NX, xrow_body, 0)
            pltpu.sync_copy(
                outb.at[pl.ds(0, NX * R), :],
                tbl.at[pl.ds(_OFFS[combo] + xb * R, NX * R), :],
            )
            return carry

        lax.fori_loop(0, NB, block_body, 0)


@functools.partial(
    pl.kernel,
    mesh=plsc.VectorSubcoreMesh(core_axis_name="c", subcore_axis_name="s"),
    out_type=jax.ShapeDtypeStruct((24, N_PTS), jnp.float32),
    compiler_params=pltpu.CompilerParams(
        needs_layout_passes=False, use_tc_tiling_on_sc=False
    ),
    scratch_types=[
        pltpu.VMEM((3 * C,), jnp.float32),
        pltpu.VMEM((12 * C,), jnp.float32),
        pltpu.VMEM((12 * C,), jnp.int32),
        pltpu.VMEM((C, 8), jnp.float32),
        pltpu.VMEM((C, 8), jnp.float32),
        pltpu.VMEM((24, C), jnp.float32),
        pltpu.VMEM((2 * LANES,), jnp.float32),
        pltpu.SemaphoreType.DMA,
        pltpu.SemaphoreType.DMA,
    ],
)
def _sc_plane_kernel(x1d, tbl, par, out_hbm, xq, fr, idxr,
                     rows_a, rows_b, outb, parv, sem_a, sem_b):
    wid = lax.axis_index("s") * NC + lax.axis_index("c")
    pltpu.sync_copy(par, parv)
    bv = parv[pl.ds(0, LANES)]
    inv = parv[pl.ds(LANES, LANES)]
    iota = lax.iota(jnp.int32, LANES)

    def chunk_body(ch, carry):
        base = wid * NPW + ch * C
        for a in range(3):
            pltpu.sync_copy(x1d.at[pl.ds(a * N_PTS + base, C)],
                            xq.at[pl.ds(a * C, C)])

        def coord_body(v, carry2):
            off16 = v * LANES
            cell = {}
            for a in range(3):
                xv = xq[pl.ds(a * C + off16, LANES)]
                xn = jnp.clip((xv + bv) * inv, 0.0, 1.0)
                for l, R in enumerate(RES):
                    p = xn * (R - 1)
                    i0 = jnp.minimum(p.astype(jnp.int32), R - 2)
                    cell[(a, l)] = i0
                    fr[pl.ds((a * 4 + l) * C + off16, LANES)] = p - i0.astype(jnp.float32)
            for pi, (a, b) in enumerate(PLANE_PAIRS):
                for l, R in enumerate(RES):
                    combo = pi * 4 + l
                    idxr[pl.ds(combo * C + off16, LANES)] = (
                        cell[(a, l)] * R + cell[(b, l)] + _OFFS[combo]
                    )
            return carry2

        lax.fori_loop(0, NV, coord_body, 0)

        bufs = (rows_a, rows_b)
        sems = (sem_a, sem_b)

        def fire(combo):
            rows, sem = bufs[combo % 2], sems[combo % 2]
            return [
                pltpu.async_copy(
                    tbl.at[idxr.at[pl.ds(combo * C + jj * GSUB, GSUB)]],
                    rows.at[pl.ds(jj * GSUB, GSUB), :],
                    sem,
                )
                for jj in range(NSUB)
            ]

        inflight = fire(0)
        for pi, (a, b) in enumerate(PLANE_PAIRS):
            for l in range(4):
                combo = pi * 4 + l
                rows = bufs[combo % 2]
                for cp in inflight:
                    cp.wait()
                if combo + 1 < 12:
                    inflight = fire(combo + 1)

                fxoff = (a * 4 + l) * C
                fyoff = (b * 4 + l) * C

                def interp_body(v, carry2, rows=rows, fxoff=fxoff,
                                fyoff=fyoff, combo=combo):
                    off16 = v * LANES
                    pt = iota + off16
                    fx = fr[pl.ds(fxoff + off16, LANES)]
                    fy = fr[pl.ds(fyoff + off16, LANES)]
                    g = [
                        plsc.load_gather(rows, [pt, jnp.full((LANES,), col, jnp.int32)])
                        for col in range(8)
                    ]
                    for ff in range(2):
                        a0 = g[ff] + fy * (g[2 + ff] - g[ff])
                        a1 = g[4 + ff] + fy * (g[6 + ff] - g[4 + ff])
                        o = a0 + fx * (a1 - a0)
                        outb[2 * combo + ff, pl.ds(off16, LANES)] = o
                    return carry2

                lax.fori_loop(0, NV, interp_body, 0, unroll=2)

        pltpu.sync_copy(outb, out_hbm.at[:, pl.ds(base, C)])
        return carry

    lax.fori_loop(0, NCHUNK, chunk_body, 0)


def kernel(x, bound,
           xy_g0, xy_g1, xy_g2, xy_g3,
           yz_g0, yz_g1, yz_g2, yz_g3,
           xz_g0, xz_g1, xz_g2, xz_g3):
    grids = [xy_g0, xy_g1, xy_g2, xy_g3,
             yz_g0, yz_g1, yz_g2, yz_g3,
             xz_g0, xz_g1, xz_g2, xz_g3]
    R2V = [g.reshape(g.shape[0], -1) for g in grids]
    tbl = _sc_table_builder(*R2V)
    b = jnp.asarray(bound, jnp.float32)
    par = jnp.concatenate([jnp.full((LANES,), b, jnp.float32),
                           jnp.full((LANES,), 0.5 / b, jnp.float32)])
    return _sc_plane_kernel(x.T.reshape(-1), tbl, par).T


# speedup vs baseline: 6.4602x; 1.0889x over previous
import functools

import jax
import jax.numpy as jnp
from jax import lax
from jax.experimental import pallas as pl
from jax.experimental.pallas import tpu as pltpu
from jax.experimental.pallas import tpu_sc as plsc

N_PTS = 524288
NC, NS, LANES = 2, 16, 16
NW = NC * NS
NPW = N_PTS // NW
C = 1024
NV = C // LANES
NCHUNK = NPW // C
GSUB = 128
NSUB = C // GSUB

RES = (128, 256, 512, 1024)
PLANE_PAIRS = ((0, 1), (0, 2), (1, 2))

_OFFS = []
_off = 0
for _pi in range(3):
    for _R in RES:
        _OFFS.append(_off)
        _off += _R * _R
TBL_ROWS = _off

BUILD_NX = {128: 4, 256: 4, 512: 2, 1024: 2}
_OUTB_ROWS = 2048


@functools.partial(
    pl.kernel,
    mesh=plsc.VectorSubcoreMesh(core_axis_name="c", subcore_axis_name="s"),
    out_type=jax.ShapeDtypeStruct((TBL_ROWS, 8), jnp.float32),
    compiler_params=pltpu.CompilerParams(
        needs_layout_passes=False, use_tc_tiling_on_sc=False
    ),
    scratch_types=[
        pltpu.VMEM((BUILD_NX[128] + 2, 2 * 128), jnp.float32),
        pltpu.VMEM((BUILD_NX[256] + 2, 2 * 256), jnp.float32),
        pltpu.VMEM((BUILD_NX[256] + 2, 2 * 256), jnp.float32),
        pltpu.VMEM((BUILD_NX[512] + 2, 2 * 512), jnp.float32),
        pltpu.VMEM((BUILD_NX[512] + 2, 2 * 512), jnp.float32),
        pltpu.VMEM((BUILD_NX[1024] + 2, 2 * 1024), jnp.float32),
        pltpu.VMEM((BUILD_NX[1024] + 2, 2 * 1024), jnp.float32),
        pltpu.VMEM((_OUTB_ROWS, 8), jnp.float32),
        pltpu.VMEM((_OUTB_ROWS, 8), jnp.float32),
        pltpu.SemaphoreType.DMA,
        pltpu.SemaphoreType.DMA,
        pltpu.SemaphoreType.DMA,
        pltpu.SemaphoreType.DMA,
    ],
)
def _sc_table_builder(*refs):
    grids = refs[:12]
    tbl = refs[12]
    strip_bufs = {128: (refs[13], refs[13]), 256: (refs[14], refs[15]),
                  512: (refs[16], refs[17]), 1024: (refs[18], refs[19])}
    outbufs = (refs[20], refs[21])
    sem_s = (refs[22], refs[23])
    sem_o = (refs[24], refs[25])
    wid = lax.axis_index("s") * NC + lax.axis_index("c")
    iota = lax.iota(jnp.int32, LANES)
    j = iota & 7
    rip = iota >> 3
    hi = (iota >> 2) & 1
    yoff = (iota >> 1) & 1
    fidx = iota & 1

    for combo in range(12):
        level = combo % 4
        R = RES[level]
        NX = BUILD_NX[R]
        XPW = R // NW
        NB = XPW // NX
        g = grids[combo]
        x0w = wid * XPW

        def strip_copy(b, buf, sem, g=g, R=R, NX=NX, x0w=x0w):
            xb = x0w + b * NX
            xs = jnp.minimum(xb, R - NX - 1)
            d = pltpu.async_copy(g.at[pl.ds(xs, NX + 1), :],
                                 buf.at[pl.ds(0, NX + 1), :], sem)
            return d, xb, xs

        descs = {0: strip_copy(0, strip_bufs[R][0], sem_s[0])}
        outd = {}
        for b in range(NB):
            par = b & 1
            d, xb, xs = descs.pop(b)
            d.wait()
            if b + 1 < NB:
                descs[b + 1] = strip_copy(b + 1, strip_bufs[R][1 - par],
                                          sem_s[1 - par])
            if b >= 2:
                outd.pop(b - 2).wait()
            strip = strip_bufs[R][par]
            outb = outbufs[par]
            xrel0 = xb - xs

            def xrow_body(xr, carry2, strip=strip, outb=outb, R=R, xrel0=xrel0):
                xi = xrel0 + xr + hi

                def yv_body(vy, carry3, strip=strip, outb=outb, R=R, xr=xr, xi=xi):
                    ci = 4 * vy + (2 * rip + 2 * yoff + fidx)
                    val = plsc.load_gather(strip, [xi, ci])
                    ri = (xr * R // 2 + vy) * 2 + rip
                    plsc.store_scatter(outb, [ri, j], val)
                    return carry3

                lax.fori_loop(0, R // 2, yv_body, 0)
                return carry2

            lax.fori_loop(0, NX, xrow_body, 0)
            outd[b] = pltpu.async_copy(
                outb.at[pl.ds(0, NX * R), :],
                tbl.at[pl.ds(_OFFS[combo] + xb * R, NX * R), :],
                sem_o[par],
            )
        for b in sorted(outd):
            outd.pop(b).wait()


@functools.partial(
    pl.kernel,
    mesh=plsc.VectorSubcoreMesh(core_axis_name="c", subcore_axis_name="s"),
    out_type=jax.ShapeDtypeStruct((24, N_PTS), jnp.float32),
    compiler_params=pltpu.CompilerParams(
        needs_layout_passes=False, use_tc_tiling_on_sc=False
    ),
    scratch_types=[
        pltpu.VMEM((3 * C,), jnp.float32),
        pltpu.VMEM((12 * C,), jnp.float32),
        pltpu.VMEM((12 * C,), jnp.int32),
        pltpu.VMEM((C, 8), jnp.float32),
        pltpu.VMEM((C, 8), jnp.float32),
        pltpu.VMEM((24, C), jnp.float32),
        pltpu.VMEM((2 * LANES,), jnp.float32),
        pltpu.SemaphoreType.DMA,
        pltpu.SemaphoreType.DMA,
    ],
)
def _sc_plane_kernel(x1d, tbl, par, out_hbm, xq, fr, idxr,
                     rows_a, rows_b, outb, parv, sem_a, sem_b):
    wid = lax.axis_index("s") * NC + lax.axis_index("c")
    pltpu.sync_copy(par, parv)
    bv = parv[pl.ds(0, LANES)]
    inv = parv[pl.ds(LANES, LANES)]
    iota = lax.iota(jnp.int32, LANES)

    def chunk_body(ch, carry):
        base = wid * NPW + ch * C
        for a in range(3):
            pltpu.sync_copy(x1d.at[pl.ds(a * N_PTS + base, C)],
                            xq.at[pl.ds(a * C, C)])

        def coord_body(v, carry2):
            off16 = v * LANES
            cell = {}
            for a in range(3):
                xv = xq[pl.ds(a * C + off16, LANES)]
                xn = jnp.clip((xv + bv) * inv, 0.0, 1.0)
                for l, R in enumerate(RES):
                    p = xn * (R - 1)
                    i0 = jnp.minimum(p.astype(jnp.int32), R - 2)
                    cell[(a, l)] = i0
                    fr[pl.ds((a * 4 + l) * C + off16, LANES)] = p - i0.astype(jnp.float32)
            for pi, (a, b) in enumerate(PLANE_PAIRS):
                for l, R in enumerate(RES):
                    combo = pi * 4 + l
                    idxr[pl.ds(combo * C + off16, LANES)] = (
                        cell[(a, l)] * R + cell[(b, l)] + _OFFS[combo]
                    )
            return carry2

        lax.fori_loop(0, NV, coord_body, 0)

        bufs = (rows_a, rows_b)
        sems = (sem_a, sem_b)

        def fire(combo):
            rows, sem = bufs[combo % 2], sems[combo % 2]
            return [
                pltpu.async_copy(
                    tbl.at[idxr.at[pl.ds(combo * C + jj * GSUB, GSUB)]],
                    rows.at[pl.ds(jj * GSUB, GSUB), :],
                    sem,
                )
                for jj in range(NSUB)
            ]

        inflight = fire(0)
        for pi, (a, b) in enumerate(PLANE_PAIRS):
            for l in range(4):
                combo = pi * 4 + l
                rows = bufs[combo % 2]
                for cp in inflight:
                    cp.wait()
                if combo + 1 < 12:
                    inflight = fire(combo + 1)

                fxoff = (a * 4 + l) * C
                fyoff = (b * 4 + l) * C

                def interp_body(v, carry2, rows=rows, fxoff=fxoff,
                                fyoff=fyoff, combo=combo):
                    off16 = v * LANES
                    pt = iota + off16
                    fx = fr[pl.ds(fxoff + off16, LANES)]
                    fy = fr[pl.ds(fyoff + off16, LANES)]
                    g = [
                        plsc.load_gather(rows, [pt, jnp.full((LANES,), col, jnp.int32)])
                        for col in range(8)
                    ]
                    for ff in range(2):
                        a0 = g[ff] + fy * (g[2 + ff] - g[ff])
                        a1 = g[4 + ff] + fy * (g[6 + ff] - g[4 + ff])
                        o = a0 + fx * (a1 - a0)
                        outb[2 * combo + ff, pl.ds(off16, LANES)] = o
                    return carry2

                lax.fori_loop(0, NV, interp_body, 0, unroll=2)

        pltpu.sync_copy(outb, out_hbm.at[:, pl.ds(base, C)])
        return carry

    lax.fori_loop(0, NCHUNK, chunk_body, 0)


def kernel(x, bound,
           xy_g0, xy_g1, xy_g2, xy_g3,
           yz_g0, yz_g1, yz_g2, yz_g3,
           xz_g0, xz_g1, xz_g2, xz_g3):
    grids = [xy_g0, xy_g1, xy_g2, xy_g3,
             yz_g0, yz_g1, yz_g2, yz_g3,
             xz_g0, xz_g1, xz_g2, xz_g3]
    R2V = [g.reshape(g.shape[0], -1) for g in grids]
    tbl = _sc_table_builder(*R2V)
    b = jnp.asarray(bound, jnp.float32)
    par = jnp.concatenate([jnp.full((LANES,), b, jnp.float32),
                           jnp.full((LANES,), 0.5 / b, jnp.float32)])
    return _sc_plane_kernel(x.T.reshape(-1), tbl, par).T


# speedup vs baseline: 7.3555x; 1.1386x over previous
import functools

import jax
import jax.numpy as jnp
from jax import lax
from jax.experimental import pallas as pl
from jax.experimental.pallas import tpu as pltpu
from jax.experimental.pallas import tpu_sc as plsc

N_PTS = 524288
NC, NS, LANES = 2, 16, 16
NW = NC * NS
NPW = N_PTS // NW
C = 1024
NV = C // LANES
NCHUNK = NPW // C
GSUB = 128
NSUB = C // GSUB

RES = (128, 256, 512, 1024)
PLANE_PAIRS = ((0, 1), (0, 2), (1, 2))

_OFFS = []
_off = 0
for _pi in range(3):
    for _R in RES:
        _OFFS.append(_off)
        _off += _R * _R
TBL_ROWS = _off

BUILD_NX = {128: 4, 256: 4, 512: 2, 1024: 2}
_OUTB_ROWS = 2048


@functools.partial(
    pl.kernel,
    mesh=plsc.VectorSubcoreMesh(core_axis_name="c", subcore_axis_name="s"),
    out_type=jax.ShapeDtypeStruct((TBL_ROWS, 8), jnp.float32),
    compiler_params=pltpu.CompilerParams(
        needs_layout_passes=False, use_tc_tiling_on_sc=False
    ),
    scratch_types=[
        pltpu.VMEM((BUILD_NX[128] + 2, 2 * 128), jnp.float32),
        pltpu.VMEM((BUILD_NX[256] + 2, 2 * 256), jnp.float32),
        pltpu.VMEM((BUILD_NX[256] + 2, 2 * 256), jnp.float32),
        pltpu.VMEM((BUILD_NX[512] + 2, 2 * 512), jnp.float32),
        pltpu.VMEM((BUILD_NX[512] + 2, 2 * 512), jnp.float32),
        pltpu.VMEM((BUILD_NX[1024] + 2, 2 * 1024), jnp.float32),
        pltpu.VMEM((BUILD_NX[1024] + 2, 2 * 1024), jnp.float32),
        pltpu.VMEM((_OUTB_ROWS, 8), jnp.float32),
        pltpu.VMEM((_OUTB_ROWS, 8), jnp.float32),
        pltpu.SemaphoreType.DMA,
        pltpu.SemaphoreType.DMA,
        pltpu.SemaphoreType.DMA,
        pltpu.SemaphoreType.DMA,
    ],
)
def _sc_table_builder(*refs):
    grids = refs[:12]
    tbl = refs[12]
    strip_bufs = {128: (refs[13], refs[13]), 256: (refs[14], refs[15]),
                  512: (refs[16], refs[17]), 1024: (refs[18], refs[19])}
    outbufs = (refs[20], refs[21])
    sem_s = (refs[22], refs[23])
    sem_o = (refs[24], refs[25])
    wid = lax.axis_index("s") * NC + lax.axis_index("c")
    iota = lax.iota(jnp.int32, LANES)
    j = iota & 7
    rip = iota >> 3
    hi = (iota >> 2) & 1
    yoff = (iota >> 1) & 1
    fidx = iota & 1

    for combo in range(12):
        level = combo % 4
        R = RES[level]
        NX = BUILD_NX[R]
        XPW = R // NW
        NB = XPW // NX
        g = grids[combo]
        x0w = wid * XPW

        def strip_copy(b, buf, sem, g=g, R=R, NX=NX, x0w=x0w):
            xb = x0w + b * NX
            xs = jnp.minimum(xb, R - NX - 1)
            d = pltpu.async_copy(g.at[pl.ds(xs, NX + 1), :],
                                 buf.at[pl.ds(0, NX + 1), :], sem)
            return d, xb, xs

        descs = {0: strip_copy(0, strip_bufs[R][0], sem_s[0])}
        outd = {}
        for b in range(NB):
            par = b & 1
            d, xb, xs = descs.pop(b)
            d.wait()
            if b + 1 < NB:
                descs[b + 1] = strip_copy(b + 1, strip_bufs[R][1 - par],
                                          sem_s[1 - par])
            if b >= 2:
                outd.pop(b - 2).wait()
            strip = strip_bufs[R][par]
            outb = outbufs[par]
            xrel0 = xb - xs

            def xrow_body(xr, carry2, strip=strip, outb=outb, R=R, xrel0=xrel0):
                xi = xrel0 + xr + hi

                def yv_body(vy, carry3, strip=strip, outb=outb, R=R, xr=xr, xi=xi):
                    ci = 4 * vy + (2 * rip + 2 * yoff + fidx)
                    val = plsc.load_gather(strip, [xi, ci])
                    ri = (xr * R // 2 + vy) * 2 + rip
                    plsc.store_scatter(outb, [ri, j], val)
                    return carry3

                lax.fori_loop(0, R // 2, yv_body, 0)
                return carry2

            lax.fori_loop(0, NX, xrow_body, 0)
            outd[b] = pltpu.async_copy(
                outb.at[pl.ds(0, NX * R), :],
                tbl.at[pl.ds(_OFFS[combo] + xb * R, NX * R), :],
                sem_o[par],
            )
        for b in sorted(outd):
            outd.pop(b).wait()


@functools.partial(
    pl.kernel,
    mesh=plsc.VectorSubcoreMesh(core_axis_name="c", subcore_axis_name="s"),
    out_type=jax.ShapeDtypeStruct((24, N_PTS), jnp.float32),
    compiler_params=pltpu.CompilerParams(
        needs_layout_passes=False, use_tc_tiling_on_sc=False
    ),
    scratch_types=[
        pltpu.VMEM((3 * C,), jnp.float32),
        pltpu.VMEM((12 * C,), jnp.float32),
        pltpu.VMEM((12 * C,), jnp.int32),
        pltpu.VMEM((C, 8), jnp.float32),
        pltpu.VMEM((C, 8), jnp.float32),
        pltpu.VMEM((C, 8), jnp.float32),
        pltpu.VMEM((24, C), jnp.float32),
        pltpu.VMEM((2 * LANES,), jnp.float32),
        pltpu.SemaphoreType.DMA,
        pltpu.SemaphoreType.DMA,
        pltpu.SemaphoreType.DMA,
    ],
)
def _sc_plane_kernel(x1d, tbl, par, out_hbm, xq, fr, idxr,
                     rows_a, rows_b, rows_c, outb, parv, sem_a, sem_b, sem_c):
    wid = lax.axis_index("s") * NC + lax.axis_index("c")
    pltpu.sync_copy(par, parv)
    bv = parv[pl.ds(0, LANES)]
    inv = parv[pl.ds(LANES, LANES)]
    iota = lax.iota(jnp.int32, LANES)

    def chunk_body(ch, carry):
        base = wid * NPW + ch * C
        for a in range(3):
            pltpu.sync_copy(x1d.at[pl.ds(a * N_PTS + base, C)],
                            xq.at[pl.ds(a * C, C)])

        def coord_body(v, carry2):
            off16 = v * LANES
            cell = {}
            for a in range(3):
                xv = xq[pl.ds(a * C + off16, LANES)]
                xn = jnp.clip((xv + bv) * inv, 0.0, 1.0)
                for l, R in enumerate(RES):
                    p = xn * (R - 1)
                    i0 = jnp.minimum(p.astype(jnp.int32), R - 2)
                    cell[(a, l)] = i0
                    fr[pl.ds((a * 4 + l) * C + off16, LANES)] = p - i0.astype(jnp.float32)
            for pi, (a, b) in enumerate(PLANE_PAIRS):
                for l, R in enumerate(RES):
                    combo = pi * 4 + l
                    idxr[pl.ds(combo * C + off16, LANES)] = (
                        cell[(a, l)] * R + cell[(b, l)] + _OFFS[combo]
                    )
            return carry2

        lax.fori_loop(0, NV, coord_body, 0)

        bufs = (rows_a, rows_b, rows_c)
        sems = (sem_a, sem_b, sem_c)

        def fire(combo):
            rows, sem = bufs[combo % 3], sems[combo % 3]
            return [
                pltpu.async_copy(
                    tbl.at[idxr.at[pl.ds(combo * C + jj * GSUB, GSUB)]],
                    rows.at[pl.ds(jj * GSUB, GSUB), :],
                    sem,
                )
                for jj in range(NSUB)
            ]

        inflight = {0: fire(0), 1: fire(1)}
        for pi, (a, b) in enumerate(PLANE_PAIRS):
            for l in range(4):
                combo = pi * 4 + l
                rows = bufs[combo % 3]
                for cp in inflight.pop(combo):
                    cp.wait()
                if combo + 2 < 12:
                    inflight[combo + 2] = fire(combo + 2)

                fxoff = (a * 4 + l) * C
                fyoff = (b * 4 + l) * C

                def interp_body(v, carry2, rows=rows, fxoff=fxoff,
                                fyoff=fyoff, combo=combo):
                    off16 = v * LANES
                    pt = iota + off16
                    fx = fr[pl.ds(fxoff + off16, LANES)]
                    fy = fr[pl.ds(fyoff + off16, LANES)]
                    g = [
                        plsc.load_gather(rows, [pt, jnp.full((LANES,), col, jnp.int32)])
                        for col in range(8)
                    ]
                    for ff in range(2):
                        a0 = g[ff] + fy * (g[2 + ff] - g[ff])
                        a1 = g[4 + ff] + fy * (g[6 + ff] - g[4 + ff])
                        o = a0 + fx * (a1 - a0)
                        outb[2 * combo + ff, pl.ds(off16, LANES)] = o
                    return carry2

                lax.fori_loop(0, NV, interp_body, 0, unroll=2)

        pltpu.sync_copy(outb, out_hbm.at[:, pl.ds(base, C)])
        return carry

    lax.fori_loop(0, NCHUNK, chunk_body, 0)


def kernel(x, bound,
           xy_g0, xy_g1, xy_g2, xy_g3,
           yz_g0, yz_g1, yz_g2, yz_g3,
           xz_g0, xz_g1, xz_g2, xz_g3):
    grids = [xy_g0, xy_g1, xy_g2, xy_g3,
             yz_g0, yz_g1, yz_g2, yz_g3,
             xz_g0, xz_g1, xz_g2, xz_g3]
    R2V = [g.reshape(g.shape[0], -1) for g in grids]
    tbl = _sc_table_builder(*R2V)
    b = jnp.asarray(bound, jnp.float32)
    par = jnp.concatenate([jnp.full((LANES,), b, jnp.float32),
                           jnp.full((LANES,), 0.5 / b, jnp.float32)])
    return _sc_plane_kernel(x.T.reshape(-1), tbl, par).T
